# baseline (speedup 1.0000x reference)
import functools

import jax
import jax.numpy as jnp
from jax import lax
from jax.experimental import pallas as pl
from jax.experimental.pallas import tpu as pltpu
from jax.experimental.pallas import tpu_sc as plsc

K_NN = 16
M_BLK = 256
TBL_W = 128

SC_CORES = 2
SC_SUBCORES = 16
SC_WORKERS = SC_CORES * SC_SUBCORES
SC_CHUNK = 128


def _stage_a_body(nblk, n, xyz_ref, xyzT_ref, feat_ref, wqT_ref, wkT_ref,
                  wvT_ref, knn_ref, q_ref, kf_ref, v_ref):
    b = pl.program_id(0)
    xb = xyz_ref[...]
    xT = xyzT_ref[0]
    fb = feat_ref[...]

    mm = jnp.dot(xb, xT, preferred_element_type=jnp.float32)
    rowsq = jnp.sum(xb * xb, axis=1, keepdims=True)
    colsq = jnp.sum(xT * xT, axis=0, keepdims=True)
    d = -2.0 * mm
    d = d + rowsq
    d = d + colsq

    m_rows = d.shape[0]
    iota = lax.broadcasted_iota(jnp.int32, (m_rows, n), 1)
    cols = []
    for _ in range(K_NN):
        mn = jnp.min(d, axis=1, keepdims=True)
        am = jnp.min(jnp.where(d <= mn, iota, n), axis=1, keepdims=True)
        cols.append(am)
        d = jnp.where(iota == am, jnp.inf, d)
    knn = jnp.concatenate(cols, axis=1)
    knn_ref[...] = knn + b * n

    q_ref[...] = jnp.dot(fb, wqT_ref[...], preferred_element_type=jnp.float32)
    kf_ref[...] = jnp.dot(fb, wkT_ref[...], preferred_element_type=jnp.float32)
    v_ref[...] = jnp.dot(fb, wvT_ref[...], preferred_element_type=jnp.float32)


def _stage_a(xyz2p, xyzTp, feat2, wqT, wkT, wvT, B, N, dim):
    nblk = N // M_BLK
    grid = (B, nblk)
    bn = B * N
    row_spec = lambda w: pl.BlockSpec((M_BLK, w), lambda b, i: (b * nblk + i, 0))
    full2 = lambda a, c: pl.BlockSpec((a, c), lambda b, i: (0, 0))
    return pl.pallas_call(
        functools.partial(_stage_a_body, nblk, N),
        grid=grid,
        in_specs=[
            row_spec(8),
            pl.BlockSpec((1, 8, N), lambda b, i: (b, 0, 0)),
            row_spec(dim),
            full2(dim, dim), full2(dim, dim), full2(dim, dim),
        ],
        out_specs=[
            row_spec(K_NN),
            row_spec(dim), row_spec(dim), row_spec(dim),
        ],
        out_shape=[
            jax.ShapeDtypeStruct((bn, K_NN), jnp.int32),
            jax.ShapeDtypeStruct((bn, dim), jnp.float32),
            jax.ShapeDtypeStruct((bn, dim), jnp.float32),
            jax.ShapeDtypeStruct((bn, dim), jnp.float32),
        ],
    )(xyz2p, xyzTp, feat2, wqT, wkT, wvT)


def _sc_gather(table, idx_flat):
    tot = idx_flat.shape[0]
    per_w = tot // SC_WORKERS
    nch = per_w // SC_CHUNK
    mesh = plsc.VectorSubcoreMesh(core_axis_name="c", subcore_axis_name="s")

    @functools.partial(
        pl.kernel,
        mesh=mesh,
        out_type=jax.ShapeDtypeStruct((tot, TBL_W), jnp.float32),
        scratch_types=[
            pltpu.VMEM((per_w,), jnp.int32),
            pltpu.VMEM((SC_CHUNK, TBL_W), jnp.float32),
            pltpu.VMEM((SC_CHUNK, TBL_W), jnp.float32),
            pltpu.SemaphoreType.DMA,
            pltpu.SemaphoreType.DMA,
        ],
    )
    def k(table_hbm, idx_hbm, out_hbm, idx_v, buf0, buf1, sem0, sem1):
        wid = lax.axis_index("s") * SC_CORES + lax.axis_index("c")
        base = wid * per_w
        pltpu.sync_copy(idx_hbm.at[pl.ds(base, per_w)], idx_v)

        def body(j, carry):
            o0 = pl.multiple_of(j * SC_CHUNK, 8)
            pltpu.async_copy(
                table_hbm.at[idx_v.at[pl.ds(o0, SC_CHUNK)]], buf0, sem0).wait()
            pltpu.sync_copy(buf0, out_hbm.at[pl.ds(base + o0, SC_CHUNK)])
            return carry

        lax.fori_loop(0, nch, body, 0)

    return k(table, idx_flat)


def _stage_c_body(g_ref, q_ref, xyzp_ref, feat_ref, pw1_ref, pb1_ref,
                  pw2_ref, pb2_ref, aw1_ref, ab1_ref, aw2_ref, ab2_ref,
                  lwT_ref, lb_ref, out_ref):
    m = q_ref.shape[0]
    dim = q_ref.shape[1]
    mk = m * K_NN
    g = g_ref[...]
    kv = lax.bitcast_convert_type(g[:, 0:dim], jnp.int32)
    kf = lax.bitcast_convert_type(
        lax.shift_left(kv, jnp.int32(16)), jnp.float32)
    v = lax.bitcast_convert_type(
        lax.bitwise_and(kv, jnp.int32(-65536)), jnp.float32)
    xg = g[:, dim:dim + 16]

    xc = xyzp_ref[...]
    rel = jnp.broadcast_to(
        xc.reshape(m, 1, 16), (m, K_NN, 16)).reshape(mk, 16) - xg
    h = jnp.maximum(
        jnp.dot(rel, pw1_ref[...], preferred_element_type=jnp.float32)
        + pb1_ref[...], 0.0)
    pos = jnp.dot(h, pw2_ref[...],
                  preferred_element_type=jnp.float32) + pb2_ref[...]

    q = q_ref[...]
    qb = jnp.broadcast_to(q.reshape(m, 1, dim), (m, K_NN, dim)).reshape(mk, dim)
    a = qb - kf + pos
    a = jnp.maximum(
        jnp.dot(a, aw1_ref[...], preferred_element_type=jnp.float32)
        + ab1_ref[...], 0.0)
    logits = (jnp.dot(a, aw2_ref[...], preferred_element_type=jnp.float32)
              + ab2_ref[...]) * (1.0 / jnp.sqrt(jnp.float32(dim)))

    l3 = logits.reshape(m, K_NN, dim)
    mx = jnp.max(l3, axis=1, keepdims=True)
    e = jnp.exp(l3 - mx)
    s = jnp.sum(e, axis=1, keepdims=True)
    attn = e / s
    wv = (v + pos).reshape(m, K_NN, dim)
    o = jnp.sum(attn * wv, axis=1)

    out_ref[...] = (jnp.dot(o, lwT_ref[...], preferred_element_type=jnp.float32)
                    + lb_ref[...]) + feat_ref[...]


def _stage_c(g, q2, xyzp, feat2, pw1, pb1, pw2, pb2, aw1, ab1, aw2, ab2,
             lwT, lb, dim):
    bn = q2.shape[0]
    nblk = bn // M_BLK
    attn_hid = aw1.shape[1]
    row = lambda w: pl.BlockSpec((M_BLK, w), lambda i: (i, 0))
    full = lambda a, c: pl.BlockSpec((a, c), lambda i: (0, 0))
    return pl.pallas_call(
        _stage_c_body,
        grid=(nblk,),
        in_specs=[
            pl.BlockSpec((M_BLK * K_NN, TBL_W), lambda i: (i, 0)),
            row(dim), row(16), row(dim),
            full(16, dim), full(1, dim),
            full(dim, dim), full(1, dim),
            full(dim, attn_hid), full(1, attn_hid),
            full(attn_hid, dim), full(1, dim),
            full(dim, dim), full(1, dim),
        ],
        out_specs=row(dim),
        out_shape=jax.ShapeDtypeStruct((bn, dim), jnp.float32),
    )(g, q2, xyzp, feat2, pw1, pb1, pw2, pb2, aw1, ab1, aw2, ab2, lwT, lb)


def kernel(xyz, feature, Wq, Wk, Wv, pe_w1, pe_b1, pe_w2, pe_b2,
           am_w1, am_b1, am_w2, am_b2, lf_w, lf_b):
    B, N, _ = xyz.shape
    dim = feature.shape[-1]
    bn = B * N

    xyz2 = xyz.reshape(bn, 3)
    xyz2p8 = jnp.pad(xyz2, ((0, 0), (0, 5)))
    xyzTp = jnp.pad(jnp.transpose(xyz, (0, 2, 1)), ((0, 0), (0, 5), (0, 0)))
    feat2 = feature.reshape(bn, dim)

    knn, q2, kf2, v2 = _stage_a(
        xyz2p8, xyzTp, feat2, Wq.T, Wk.T, Wv.T, B, N, dim)

    xyzp16 = jnp.pad(xyz2, ((0, 0), (0, 13)))
    kfi = lax.bitcast_convert_type(
        kf2.astype(jnp.bfloat16), jnp.uint16).astype(jnp.uint32)
    vi = lax.bitcast_convert_type(
        v2.astype(jnp.bfloat16), jnp.uint16).astype(jnp.uint32)
    kv = lax.bitcast_convert_type(kfi | (vi << jnp.uint32(16)), jnp.float32)
    table = jnp.concatenate(
        [kv, xyzp16, jnp.zeros((bn, TBL_W - 80), jnp.float32)], axis=1)
    g = _sc_gather(table, knn.reshape(bn * K_NN))

    pw1 = jnp.zeros((16, pe_w1.shape[0]), jnp.float32).at[:3, :].set(pe_w1.T)
    out2 = _stage_c(
        g, q2, xyzp16, feat2,
        pw1, pe_b1.reshape(1, -1), pe_w2.T, pe_b2.reshape(1, -1),
        am_w1.T, am_b1.reshape(1, -1), am_w2.T, am_b2.reshape(1, -1),
        lf_w.T, lf_b.reshape(1, -1), dim)
    return out2.reshape(B, N, dim)


# speedup vs baseline: 1.3580x; 1.3580x over previous
import functools

import jax
import jax.numpy as jnp
from jax import lax
from jax.experimental import pallas as pl
from jax.experimental.pallas import tpu as pltpu
from jax.experimental.pallas import tpu_sc as plsc

K_NN = 16
M_BLK = 256
TBL_W = 128

SC_CORES = 2
SC_SUBCORES = 16
SC_WORKERS = SC_CORES * SC_SUBCORES
SC_CHUNK = 128


def _stage_a_body(nblk, n, xyz_ref, xyzT_ref, feat_ref, wqT_ref, wkT_ref,
                  wvT_ref, knn_ref, q_ref, kf_ref, v_ref):
    b = pl.program_id(0)
    xb = xyz_ref[...]
    xT = xyzT_ref[0]
    fb = feat_ref[...]

    mm = jnp.dot(xb, xT, preferred_element_type=jnp.float32)
    rowsq = jnp.sum(xb * xb, axis=1, keepdims=True)
    colsq = jnp.sum(xT * xT, axis=0, keepdims=True)
    d = -2.0 * mm
    d = d + rowsq
    d = d + colsq

    m_rows = d.shape[0]
    ngrp = n // 128
    inf = jnp.float32(jnp.inf)
    big = jnp.int32(1 << 30)
    iota_l = lax.broadcasted_iota(jnp.int32, (m_rows, 128), 1)
    a1 = d[:, 0:128]
    c1 = iota_l
    a2 = jnp.full((m_rows, 128), inf, jnp.float32)
    c2 = jnp.full((m_rows, 128), big, jnp.int32)
    a3 = a2
    c3 = c2
    for s in range(1, ngrp):
        v = d[:, s * 128:(s + 1) * 128]
        cc = iota_l + s * 128
        l1 = v < a1
        l2 = v < a2
        l3 = v < a3
        a3n = jnp.where(l3, jnp.where(l2, a2, v), a3)
        c3n = jnp.where(l3, jnp.where(l2, c2, cc), c3)
        a2n = jnp.where(l2, jnp.where(l1, a1, v), a2)
        c2n = jnp.where(l2, jnp.where(l1, c1, cc), c2)
        a1 = jnp.where(l1, v, a1)
        c1 = jnp.where(l1, cc, c1)
        a2, a3, c2, c3 = a2n, a3n, c2n, c3n
    S = jnp.concatenate([a1, a2, a3], axis=1)
    CC = jnp.concatenate([c1, c2, c3], axis=1)
    cols = []
    for _ in range(K_NN):
        mn = jnp.min(S, axis=1, keepdims=True)
        col = jnp.min(jnp.where(S <= mn, CC, big), axis=1, keepdims=True)
        cols.append(col)
        S = jnp.where(CC == col, inf, S)
    knn = jnp.concatenate(cols, axis=1)
    knn_ref[...] = knn + b * n

    q_ref[...] = jnp.dot(fb, wqT_ref[...], preferred_element_type=jnp.float32)
    kf_ref[...] = jnp.dot(fb, wkT_ref[...], preferred_element_type=jnp.float32)
    v_ref[...] = jnp.dot(fb, wvT_ref[...], preferred_element_type=jnp.float32)


def _stage_a(xyz2p, xyzTp, feat2, wqT, wkT, wvT, B, N, dim):
    nblk = N // M_BLK
    grid = (B, nblk)
    bn = B * N
    row_spec = lambda w: pl.BlockSpec((M_BLK, w), lambda b, i: (b * nblk + i, 0))
    full2 = lambda a, c: pl.BlockSpec((a, c), lambda b, i: (0, 0))
    return pl.pallas_call(
        functools.partial(_stage_a_body, nblk, N),
        grid=grid,
        in_specs=[
            row_spec(8),
            pl.BlockSpec((1, 8, N), lambda b, i: (b, 0, 0)),
            row_spec(dim),
            full2(dim, dim), full2(dim, dim), full2(dim, dim),
        ],
        out_specs=[
            row_spec(K_NN),
            row_spec(dim), row_spec(dim), row_spec(dim),
        ],
        out_shape=[
            jax.ShapeDtypeStruct((bn, K_NN), jnp.int32),
            jax.ShapeDtypeStruct((bn, dim), jnp.float32),
            jax.ShapeDtypeStruct((bn, dim), jnp.float32),
            jax.ShapeDtypeStruct((bn, dim), jnp.float32),
        ],
    )(xyz2p, xyzTp, feat2, wqT, wkT, wvT)


def _sc_gather(table, idx_flat):
    tot = idx_flat.shape[0]
    per_w = tot // SC_WORKERS
    nch = per_w // SC_CHUNK
    mesh = plsc.VectorSubcoreMesh(core_axis_name="c", subcore_axis_name="s")

    @functools.partial(
        pl.kernel,
        mesh=mesh,
        out_type=jax.ShapeDtypeStruct((tot, TBL_W), jnp.float32),
        scratch_types=[
            pltpu.VMEM((per_w,), jnp.int32),
            pltpu.VMEM((SC_CHUNK, TBL_W), jnp.float32),
            pltpu.VMEM((SC_CHUNK, TBL_W), jnp.float32),
            pltpu.SemaphoreType.DMA,
            pltpu.SemaphoreType.DMA,
        ],
    )
    def k(table_hbm, idx_hbm, out_hbm, idx_v, buf0, buf1, sem0, sem1):
        wid = lax.axis_index("s") * SC_CORES + lax.axis_index("c")
        base = wid * per_w
        pltpu.sync_copy(idx_hbm.at[pl.ds(base, per_w)], idx_v)

        def body(j, carry):
            o0 = pl.multiple_of(j * SC_CHUNK, 8)
            pltpu.async_copy(
                table_hbm.at[idx_v.at[pl.ds(o0, SC_CHUNK)]], buf0, sem0).wait()
            pltpu.sync_copy(buf0, out_hbm.at[pl.ds(base + o0, SC_CHUNK)])
            return carry

        lax.fori_loop(0, nch, body, 0)

    return k(table, idx_flat)


def _stage_c_body(g_ref, q_ref, xyzp_ref, feat_ref, pw1_ref, pb1_ref,
                  pw2_ref, pb2_ref, aw1_ref, ab1_ref, aw2_ref, ab2_ref,
                  lwT_ref, lb_ref, out_ref):
    m = q_ref.shape[0]
    dim = q_ref.shape[1]
    mk = m * K_NN
    g = g_ref[...]
    kv = lax.bitcast_convert_type(g[:, 0:dim], jnp.int32)
    kf = lax.bitcast_convert_type(
        lax.shift_left(kv, jnp.int32(16)), jnp.float32)
    v = lax.bitcast_convert_type(
        lax.bitwise_and(kv, jnp.int32(-65536)), jnp.float32)
    xg = g[:, dim:dim + 16]

    xc = xyzp_ref[...]
    rel = jnp.broadcast_to(
        xc.reshape(m, 1, 16), (m, K_NN, 16)).reshape(mk, 16) - xg
    h = jnp.maximum(
        jnp.dot(rel, pw1_ref[...], preferred_element_type=jnp.float32)
        + pb1_ref[...], 0.0)
    pos = jnp.dot(h, pw2_ref[...],
                  preferred_element_type=jnp.float32) + pb2_ref[...]

    q = q_ref[...]
    qb = jnp.broadcast_to(q.reshape(m, 1, dim), (m, K_NN, dim)).reshape(mk, dim)
    a = qb - kf + pos
    a = jnp.maximum(
        jnp.dot(a, aw1_ref[...], preferred_element_type=jnp.float32)
        + ab1_ref[...], 0.0)
    logits = (jnp.dot(a, aw2_ref[...], preferred_element_type=jnp.float32)
              + ab2_ref[...]) * (1.0 / jnp.sqrt(jnp.float32(dim)))

    l3 = logits.reshape(m, K_NN, dim)
    mx = jnp.max(l3, axis=1, keepdims=True)
    e = jnp.exp(l3 - mx)
    s = jnp.sum(e, axis=1, keepdims=True)
    attn = e / s
    wv = (v + pos).reshape(m, K_NN, dim)
    o = jnp.sum(attn * wv, axis=1)

    out_ref[...] = (jnp.dot(o, lwT_ref[...], preferred_element_type=jnp.float32)
                    + lb_ref[...]) + feat_ref[...]


def _stage_c(g, q2, xyzp, feat2, pw1, pb1, pw2, pb2, aw1, ab1, aw2, ab2,
             lwT, lb, dim):
    bn = q2.shape[0]
    nblk = bn // M_BLK
    attn_hid = aw1.shape[1]
    row = lambda w: pl.BlockSpec((M_BLK, w), lambda i: (i, 0))
    full = lambda a, c: pl.BlockSpec((a, c), lambda i: (0, 0))
    return pl.pallas_call(
        _stage_c_body,
        grid=(nblk,),
        in_specs=[
            pl.BlockSpec((M_BLK * K_NN, TBL_W), lambda i: (i, 0)),
            row(dim), row(16), row(dim),
            full(16, dim), full(1, dim),
            full(dim, dim), full(1, dim),
            full(dim, attn_hid), full(1, attn_hid),
            full(attn_hid, dim), full(1, dim),
            full(dim, dim), full(1, dim),
        ],
        out_specs=row(dim),
        out_shape=jax.ShapeDtypeStruct((bn, dim), jnp.float32),
    )(g, q2, xyzp, feat2, pw1, pb1, pw2, pb2, aw1, ab1, aw2, ab2, lwT, lb)


def kernel(xyz, feature, Wq, Wk, Wv, pe_w1, pe_b1, pe_w2, pe_b2,
           am_w1, am_b1, am_w2, am_b2, lf_w, lf_b):
    B, N, _ = xyz.shape
    dim = feature.shape[-1]
    bn = B * N

    xyz2 = xyz.reshape(bn, 3)
    xyz2p8 = jnp.pad(xyz2, ((0, 0), (0, 5)))
    xyzTp = jnp.pad(jnp.transpose(xyz, (0, 2, 1)), ((0, 0), (0, 5), (0, 0)))
    feat2 = feature.reshape(bn, dim)

    knn, q2, kf2, v2 = _stage_a(
        xyz2p8, xyzTp, feat2, Wq.T, Wk.T, Wv.T, B, N, dim)

    xyzp16 = jnp.pad(xyz2, ((0, 0), (0, 13)))
    kfi = lax.bitcast_convert_type(
        kf2.astype(jnp.bfloat16), jnp.uint16).astype(jnp.uint32)
    vi = lax.bitcast_convert_type(
        v2.astype(jnp.bfloat16), jnp.uint16).astype(jnp.uint32)
    kv = lax.bitcast_convert_type(kfi | (vi << jnp.uint32(16)), jnp.float32)
    table = jnp.concatenate(
        [kv, xyzp16, jnp.zeros((bn, TBL_W - 80), jnp.float32)], axis=1)
    g = _sc_gather(table, knn.reshape(bn * K_NN))

    pw1 = jnp.zeros((16, pe_w1.shape[0]), jnp.float32).at[:3, :].set(pe_w1.T)
    out2 = _stage_c(
        g, q2, xyzp16, feat2,
        pw1, pe_b1.reshape(1, -1), pe_w2.T, pe_b2.reshape(1, -1),
        am_w1.T, am_b1.reshape(1, -1), am_w2.T, am_b2.reshape(1, -1),
        lf_w.T, lf_b.reshape(1, -1), dim)
    return out2.reshape(B, N, dim)


# speedup vs baseline: 1.3880x; 1.0221x over previous
import functools

import jax
import jax.numpy as jnp
from jax import lax
from jax.experimental import pallas as pl
from jax.experimental.pallas import tpu as pltpu
from jax.experimental.pallas import tpu_sc as plsc

K_NN = 16
M_BLK = 256
TBL_W = 128

SC_CORES = 2
SC_SUBCORES = 16
SC_WORKERS = SC_CORES * SC_SUBCORES
SC_CHUNK = 128


def _stage_a_body(nblk, n, xyz_ref, xyzT_ref, feat_ref, wqT_ref, wkT_ref,
                  wvT_ref, knn_ref, q_ref, table_ref):
    b = pl.program_id(0)
    xb = xyz_ref[...]
    xT = xyzT_ref[0]
    fb = feat_ref[...]

    mm = jnp.dot(xb, xT, preferred_element_type=jnp.float32)
    rowsq = jnp.sum(xb * xb, axis=1, keepdims=True)
    colsq = jnp.sum(xT * xT, axis=0, keepdims=True)
    d = -2.0 * mm
    d = d + rowsq
    d = d + colsq

    m_rows = d.shape[0]
    ngrp = n // 128
    inf = jnp.float32(jnp.inf)
    big = jnp.int32(1 << 30)
    iota_l = lax.broadcasted_iota(jnp.int32, (m_rows, 128), 1)
    a1 = d[:, 0:128]
    c1 = iota_l
    a2 = jnp.full((m_rows, 128), inf, jnp.float32)
    c2 = jnp.full((m_rows, 128), big, jnp.int32)
    a3 = a2
    c3 = c2
    for s in range(1, ngrp):
        v = d[:, s * 128:(s + 1) * 128]
        cc = iota_l + s * 128
        l1 = v < a1
        l2 = v < a2
        l3 = v < a3
        a3n = jnp.where(l3, jnp.where(l2, a2, v), a3)
        c3n = jnp.where(l3, jnp.where(l2, c2, cc), c3)
        a2n = jnp.where(l2, jnp.where(l1, a1, v), a2)
        c2n = jnp.where(l2, jnp.where(l1, c1, cc), c2)
        a1 = jnp.where(l1, v, a1)
        c1 = jnp.where(l1, cc, c1)
        a2, a3, c2, c3 = a2n, a3n, c2n, c3n
    S = jnp.concatenate([a1, a2, a3], axis=1)
    CC = jnp.concatenate([c1, c2, c3], axis=1)
    cols = []
    for _ in range(K_NN):
        mn = jnp.min(S, axis=1, keepdims=True)
        col = jnp.min(jnp.where(S <= mn, CC, big), axis=1, keepdims=True)
        cols.append(col)
        S = jnp.where(CC == col, inf, S)
    knn = jnp.concatenate(cols, axis=1)
    knn_ref[...] = knn + b * n

    q_ref[...] = jnp.dot(fb, wqT_ref[...], preferred_element_type=jnp.float32)
    kf = jnp.dot(fb, wkT_ref[...], preferred_element_type=jnp.float32)
    v = jnp.dot(fb, wvT_ref[...], preferred_element_type=jnp.float32)
    kfi = lax.bitcast_convert_type(
        kf.astype(jnp.bfloat16), jnp.uint16).astype(jnp.uint32)
    vi = lax.bitcast_convert_type(
        v.astype(jnp.bfloat16), jnp.uint16).astype(jnp.uint32)
    kv = lax.bitcast_convert_type(kfi | (vi << jnp.uint32(16)), jnp.float32)
    table_ref[...] = jnp.concatenate(
        [kv, xb, jnp.zeros((m_rows, 56), jnp.float32)], axis=1)


def _stage_a(xyz2p, xyzTp, feat2, wqT, wkT, wvT, B, N, dim):
    nblk = N // M_BLK
    grid = (B, nblk)
    bn = B * N
    row_spec = lambda w: pl.BlockSpec((M_BLK, w), lambda b, i: (b * nblk + i, 0))
    full2 = lambda a, c: pl.BlockSpec((a, c), lambda b, i: (0, 0))
    return pl.pallas_call(
        functools.partial(_stage_a_body, nblk, N),
        grid=grid,
        in_specs=[
            row_spec(8),
            pl.BlockSpec((1, 8, N), lambda b, i: (b, 0, 0)),
            row_spec(dim),
            full2(dim, dim), full2(dim, dim), full2(dim, dim),
        ],
        out_specs=[
            row_spec(K_NN),
            row_spec(dim),
            row_spec(TBL_W),
        ],
        out_shape=[
            jax.ShapeDtypeStruct((bn, K_NN), jnp.int32),
            jax.ShapeDtypeStruct((bn, dim), jnp.float32),
            jax.ShapeDtypeStruct((bn, TBL_W), jnp.float32),
        ],
    )(xyz2p, xyzTp, feat2, wqT, wkT, wvT)


def _sc_gather(table, idx_flat):
    tot = idx_flat.shape[0]
    per_w = tot // SC_WORKERS
    nch = per_w // SC_CHUNK
    mesh = plsc.VectorSubcoreMesh(core_axis_name="c", subcore_axis_name="s")

    @functools.partial(
        pl.kernel,
        mesh=mesh,
        out_type=jax.ShapeDtypeStruct((tot, TBL_W), jnp.float32),
        scratch_types=[
            pltpu.VMEM((per_w,), jnp.int32),
            pltpu.VMEM((SC_CHUNK, TBL_W), jnp.float32),
            pltpu.VMEM((SC_CHUNK, TBL_W), jnp.float32),
            pltpu.SemaphoreType.DMA,
            pltpu.SemaphoreType.DMA,
        ],
    )
    def k(table_hbm, idx_hbm, out_hbm, idx_v, buf0, buf1, sem0, sem1):
        wid = lax.axis_index("s") * SC_CORES + lax.axis_index("c")
        base = wid * per_w
        pltpu.sync_copy(idx_hbm.at[pl.ds(base, per_w)], idx_v)

        def body(j, carry):
            o0 = pl.multiple_of(j * SC_CHUNK, 8)
            pltpu.async_copy(
                table_hbm.at[idx_v.at[pl.ds(o0, SC_CHUNK)]], buf0, sem0).wait()
            pltpu.sync_copy(buf0, out_hbm.at[pl.ds(base + o0, SC_CHUNK)])
            return carry

        lax.fori_loop(0, nch, body, 0)

    return k(table, idx_flat)


def _stage_c_body(g_ref, q_ref, xyzp_ref, feat_ref, pw1_ref, pb1_ref,
                  pw2_ref, pb2_ref, aw1_ref, ab1_ref, aw2_ref, ab2_ref,
                  lwT_ref, lb_ref, out_ref):
    m = q_ref.shape[0]
    dim = q_ref.shape[1]
    mk = m * K_NN
    g = g_ref[...]
    kv = lax.bitcast_convert_type(g[:, 0:dim], jnp.int32)
    kf = lax.bitcast_convert_type(
        lax.shift_left(kv, jnp.int32(16)), jnp.float32)
    v = lax.bitcast_convert_type(
        lax.bitwise_and(kv, jnp.int32(-65536)), jnp.float32)
    xg = g[:, dim:dim + 16]

    xc = xyzp_ref[...]
    rel = jnp.broadcast_to(
        xc.reshape(m, 1, 16), (m, K_NN, 16)).reshape(mk, 16) - xg
    h = jnp.maximum(
        jnp.dot(rel, pw1_ref[...], preferred_element_type=jnp.float32)
        + pb1_ref[...], 0.0)
    pos = jnp.dot(h, pw2_ref[...],
                  preferred_element_type=jnp.float32) + pb2_ref[...]

    q = q_ref[...]
    qb = jnp.broadcast_to(q.reshape(m, 1, dim), (m, K_NN, dim)).reshape(mk, dim)
    a = qb - kf + pos
    a = jnp.maximum(
        jnp.dot(a, aw1_ref[...], preferred_element_type=jnp.float32)
        + ab1_ref[...], 0.0)
    logits = (jnp.dot(a, aw2_ref[...], preferred_element_type=jnp.float32)
              + ab2_ref[...]) * (1.0 / jnp.sqrt(jnp.float32(dim)))

    l3 = logits.reshape(m, K_NN, dim)
    mx = jnp.max(l3, axis=1, keepdims=True)
    e = jnp.exp(l3 - mx)
    s = jnp.sum(e, axis=1, keepdims=True)
    attn = e / s
    wv = (v + pos).reshape(m, K_NN, dim)
    o = jnp.sum(attn * wv, axis=1)

    out_ref[...] = (jnp.dot(o, lwT_ref[...], preferred_element_type=jnp.float32)
                    + lb_ref[...]) + feat_ref[...]


def _stage_c(g, q2, xyzp, feat2, pw1, pb1, pw2, pb2, aw1, ab1, aw2, ab2,
             lwT, lb, dim):
    bn = q2.shape[0]
    nblk = bn // M_BLK
    attn_hid = aw1.shape[1]
    row = lambda w: pl.BlockSpec((M_BLK, w), lambda i: (i, 0))
    full = lambda a, c: pl.BlockSpec((a, c), lambda i: (0, 0))
    return pl.pallas_call(
        _stage_c_body,
        grid=(nblk,),
        in_specs=[
            pl.BlockSpec((M_BLK * K_NN, TBL_W), lambda i: (i, 0)),
            row(dim), row(16), row(dim),
            full(16, dim), full(1, dim),
            full(dim, dim), full(1, dim),
            full(dim, attn_hid), full(1, attn_hid),
            full(attn_hid, dim), full(1, dim),
            full(dim, dim), full(1, dim),
        ],
        out_specs=row(dim),
        out_shape=jax.ShapeDtypeStruct((bn, dim), jnp.float32),
    )(g, q2, xyzp, feat2, pw1, pb1, pw2, pb2, aw1, ab1, aw2, ab2, lwT, lb)


def kernel(xyz, feature, Wq, Wk, Wv, pe_w1, pe_b1, pe_w2, pe_b2,
           am_w1, am_b1, am_w2, am_b2, lf_w, lf_b):
    B, N, _ = xyz.shape
    dim = feature.shape[-1]
    bn = B * N

    xyz2 = xyz.reshape(bn, 3)
    xyz2p8 = jnp.pad(xyz2, ((0, 0), (0, 5)))
    xyzTp = jnp.pad(jnp.transpose(xyz, (0, 2, 1)), ((0, 0), (0, 5), (0, 0)))
    feat2 = feature.reshape(bn, dim)

    knn, q2, table = _stage_a(
        xyz2p8, xyzTp, feat2, Wq.T, Wk.T, Wv.T, B, N, dim)

    xyzp16 = jnp.pad(xyz2, ((0, 0), (0, 13)))
    g = _sc_gather(table, knn.reshape(bn * K_NN))

    pw1 = jnp.zeros((16, pe_w1.shape[0]), jnp.float32).at[:3, :].set(pe_w1.T)
    out2 = _stage_c(
        g, q2, xyzp16, feat2,
        pw1, pe_b1.reshape(1, -1), pe_w2.T, pe_b2.reshape(1, -1),
        am_w1.T, am_b1.reshape(1, -1), am_w2.T, am_b2.reshape(1, -1),
        lf_w.T, lf_b.reshape(1, -1), dim)
    return out2.reshape(B, N, dim)


# speedup vs baseline: 1.5734x; 1.1336x over previous
import functools

import jax
import jax.numpy as jnp
from jax import lax
from jax.experimental import pallas as pl
from jax.experimental.pallas import tpu as pltpu
from jax.experimental.pallas import tpu_sc as plsc

K_NN = 16
M_BLK = 256
TBL_W = 128

SC_CORES = 2
SC_SUBCORES = 16
SC_WORKERS = SC_CORES * SC_SUBCORES
SC_CHUNK = 128


def _stage_a_body(nblk, n, xyz_ref, xyzT_ref, feat_ref, wqT_ref, wkT_ref,
                  wvT_ref, knn_ref, q_ref, table_ref):
    b = pl.program_id(0)
    xb = xyz_ref[...]
    xT = xyzT_ref[0]
    fb = feat_ref[...]

    mm = jnp.dot(xb, xT, preferred_element_type=jnp.float32)
    rowsq = jnp.sum(xb * xb, axis=1, keepdims=True)
    colsq = jnp.sum(xT * xT, axis=0, keepdims=True)
    d = -2.0 * mm
    d = d + rowsq
    d = d + colsq

    m_rows = d.shape[0]
    ngrp = n // 128
    inf = jnp.float32(jnp.inf)
    big = jnp.int32(1 << 30)
    iota_l = lax.broadcasted_iota(jnp.int32, (m_rows, 128), 1)
    a1 = d[:, 0:128]
    c1 = iota_l
    a2 = jnp.full((m_rows, 128), inf, jnp.float32)
    c2 = jnp.full((m_rows, 128), big, jnp.int32)
    a3 = a2
    c3 = c2
    for s in range(1, ngrp):
        v = d[:, s * 128:(s + 1) * 128]
        cc = iota_l + s * 128
        l1 = v < a1
        l2 = v < a2
        l3 = v < a3
        a3n = jnp.where(l3, jnp.where(l2, a2, v), a3)
        c3n = jnp.where(l3, jnp.where(l2, c2, cc), c3)
        a2n = jnp.where(l2, jnp.where(l1, a1, v), a2)
        c2n = jnp.where(l2, jnp.where(l1, c1, cc), c2)
        a1 = jnp.where(l1, v, a1)
        c1 = jnp.where(l1, cc, c1)
        a2, a3, c2, c3 = a2n, a3n, c2n, c3n
    S = jnp.concatenate([a1, a2, a3], axis=1)
    CCf = jnp.concatenate([c1, c2, c3], axis=1).astype(jnp.float32)
    bigf = jnp.float32(1e9)
    cols = []
    for _ in range(K_NN):
        mn = jnp.min(S, axis=1, keepdims=True)
        colf = jnp.min(jnp.where(S <= mn, CCf, bigf), axis=1, keepdims=True)
        cols.append(colf)
        S = jnp.where(CCf == colf, inf, S)
    knn = jnp.concatenate(cols, axis=1).astype(jnp.int32)
    knn_ref[...] = knn + b * n

    q_ref[...] = jnp.dot(fb, wqT_ref[...], preferred_element_type=jnp.float32)
    kf = jnp.dot(fb, wkT_ref[...], preferred_element_type=jnp.float32)
    v = jnp.dot(fb, wvT_ref[...], preferred_element_type=jnp.float32)
    kfi = lax.bitcast_convert_type(
        kf.astype(jnp.bfloat16), jnp.uint16).astype(jnp.uint32)
    vi = lax.bitcast_convert_type(
        v.astype(jnp.bfloat16), jnp.uint16).astype(jnp.uint32)
    kv = lax.bitcast_convert_type(kfi | (vi << jnp.uint32(16)), jnp.float32)
    table_ref[...] = jnp.concatenate(
        [kv, xb, jnp.zeros((m_rows, 56), jnp.float32)], axis=1)


def _stage_a(xyz2p, xyzTp, feat2, wqT, wkT, wvT, B, N, dim):
    nblk = N // M_BLK
    grid = (B, nblk)
    bn = B * N
    row_spec = lambda w: pl.BlockSpec((M_BLK, w), lambda b, i: (b * nblk + i, 0))
    full2 = lambda a, c: pl.BlockSpec((a, c), lambda b, i: (0, 0))
    return pl.pallas_call(
        functools.partial(_stage_a_body, nblk, N),
        grid=grid,
        in_specs=[
            row_spec(8),
            pl.BlockSpec((1, 8, N), lambda b, i: (b, 0, 0)),
            row_spec(dim),
            full2(dim, dim), full2(dim, dim), full2(dim, dim),
        ],
        out_specs=[
            row_spec(K_NN),
            row_spec(dim),
            row_spec(TBL_W),
        ],
        out_shape=[
            jax.ShapeDtypeStruct((bn, K_NN), jnp.int32),
            jax.ShapeDtypeStruct((bn, dim), jnp.float32),
            jax.ShapeDtypeStruct((bn, TBL_W), jnp.float32),
        ],
    )(xyz2p, xyzTp, feat2, wqT, wkT, wvT)


def _sc_gather(table, idx_flat):
    tot = idx_flat.shape[0]
    per_w = tot // SC_WORKERS
    nch = per_w // SC_CHUNK
    mesh = plsc.VectorSubcoreMesh(core_axis_name="c", subcore_axis_name="s")

    @functools.partial(
        pl.kernel,
        mesh=mesh,
        out_type=jax.ShapeDtypeStruct((tot, TBL_W), jnp.float32),
        scratch_types=[
            pltpu.VMEM((per_w,), jnp.int32),
            pltpu.VMEM((SC_CHUNK, TBL_W), jnp.float32),
            pltpu.VMEM((SC_CHUNK, TBL_W), jnp.float32),
            pltpu.SemaphoreType.DMA,
            pltpu.SemaphoreType.DMA,
        ],
    )
    def k(table_hbm, idx_hbm, out_hbm, idx_v, buf0, buf1, sem0, sem1):
        wid = lax.axis_index("s") * SC_CORES + lax.axis_index("c")
        base = wid * per_w
        pltpu.sync_copy(idx_hbm.at[pl.ds(base, per_w)], idx_v)

        def body(j, carry):
            o0 = pl.multiple_of(j * SC_CHUNK, 8)
            pltpu.async_copy(
                table_hbm.at[idx_v.at[pl.ds(o0, SC_CHUNK)]], buf0, sem0).wait()
            pltpu.sync_copy(buf0, out_hbm.at[pl.ds(base + o0, SC_CHUNK)])
            return carry

        lax.fori_loop(0, nch, body, 0)

    return k(table, idx_flat)


def _stage_c_body(g_ref, q_ref, xyzp_ref, feat_ref, pw1_ref, pb1_ref,
                  pw2_ref, pb2_ref, aw1_ref, ab1_ref, aw2_ref, ab2_ref,
                  lwT_ref, lb_ref, out_ref):
    m = q_ref.shape[0]
    dim = q_ref.shape[1]
    mk = m * K_NN
    g = g_ref[...]
    kv = lax.bitcast_convert_type(g[:, :, 0:dim], jnp.int32)
    kf = lax.bitcast_convert_type(
        lax.shift_left(kv, jnp.int32(16)), jnp.float32)
    v = lax.bitcast_convert_type(
        lax.bitwise_and(kv, jnp.int32(-65536)), jnp.float32)
    xg = g[:, :, dim:dim + 16]

    xc = xyzp_ref[...]
    bf = jnp.bfloat16
    rel = (xc[None, :, :] - xg).reshape(mk, 16)
    h = jnp.maximum(
        jnp.dot(rel, pw1_ref[...], preferred_element_type=jnp.float32)
        + pb1_ref[...], 0.0)
    pos = jnp.dot(h.astype(bf), pw2_ref[...].astype(bf),
                  preferred_element_type=jnp.float32) + pb2_ref[...]

    q = q_ref[...]
    pos3 = pos.reshape(K_NN, m, dim)
    a = (q[None, :, :] - kf + pos3).reshape(mk, dim)
    a = jnp.maximum(
        jnp.dot(a.astype(bf), aw1_ref[...].astype(bf),
                preferred_element_type=jnp.float32)
        + ab1_ref[...], 0.0)
    logits = (jnp.dot(a.astype(bf), aw2_ref[...].astype(bf),
                      preferred_element_type=jnp.float32)
              + ab2_ref[...]) * (1.0 / jnp.sqrt(jnp.float32(dim)))

    l3 = logits.reshape(K_NN, m, dim)
    mx = jnp.max(l3, axis=0, keepdims=True)
    e = jnp.exp(l3 - mx)
    s = jnp.sum(e, axis=0, keepdims=True)
    attn = e / s
    wv = v + pos3
    o = jnp.sum(attn * wv, axis=0)

    out_ref[...] = (jnp.dot(o, lwT_ref[...], preferred_element_type=jnp.float32)
                    + lb_ref[...]) + feat_ref[...]


def _stage_c(g, q2, xyzp, feat2, pw1, pb1, pw2, pb2, aw1, ab1, aw2, ab2,
             lwT, lb, dim):
    bn = q2.shape[0]
    nblk = bn // M_BLK
    attn_hid = aw1.shape[1]
    row = lambda w: pl.BlockSpec((M_BLK, w), lambda i: (i, 0))
    full = lambda a, c: pl.BlockSpec((a, c), lambda i: (0, 0))
    return pl.pallas_call(
        _stage_c_body,
        grid=(nblk,),
        in_specs=[
            pl.BlockSpec((K_NN, M_BLK, TBL_W), lambda i: (0, i, 0)),
            row(dim), row(16), row(dim),
            full(16, dim), full(1, dim),
            full(dim, dim), full(1, dim),
            full(dim, attn_hid), full(1, attn_hid),
            full(attn_hid, dim), full(1, dim),
            full(dim, dim), full(1, dim),
        ],
        out_specs=row(dim),
        out_shape=jax.ShapeDtypeStruct((bn, dim), jnp.float32),
    )(g, q2, xyzp, feat2, pw1, pb1, pw2, pb2, aw1, ab1, aw2, ab2, lwT, lb)


def kernel(xyz, feature, Wq, Wk, Wv, pe_w1, pe_b1, pe_w2, pe_b2,
           am_w1, am_b1, am_w2, am_b2, lf_w, lf_b):
    B, N, _ = xyz.shape
    dim = feature.shape[-1]
    bn = B * N

    xyz2 = xyz.reshape(bn, 3)
    xyz2p8 = jnp.pad(xyz2, ((0, 0), (0, 5)))
    xyzTp = jnp.pad(jnp.transpose(xyz, (0, 2, 1)), ((0, 0), (0, 5), (0, 0)))
    feat2 = feature.reshape(bn, dim)

    knn, q2, table = _stage_a(
        xyz2p8, xyzTp, feat2, Wq.T, Wk.T, Wv.T, B, N, dim)

    xyzp16 = jnp.pad(xyz2, ((0, 0), (0, 13)))
    g = _sc_gather(table, knn.T.reshape(bn * K_NN))
    g = g.reshape(K_NN, bn, TBL_W)

    pw1 = jnp.zeros((16, pe_w1.shape[0]), jnp.float32).at[:3, :].set(pe_w1.T)
    out2 = _stage_c(
        g, q2, xyzp16, feat2,
        pw1, pe_b1.reshape(1, -1), pe_w2.T, pe_b2.reshape(1, -1),
        am_w1.T, am_b1.reshape(1, -1), am_w2.T, am_b2.reshape(1, -1),
        lf_w.T, lf_b.reshape(1, -1), dim)
    return out2.reshape(B, N, dim)


# speedup vs baseline: 1.6668x; 1.0594x over previous
import functools

import jax
import jax.numpy as jnp
from jax import lax
from jax.experimental import pallas as pl
from jax.experimental.pallas import tpu as pltpu
from jax.experimental.pallas import tpu_sc as plsc

K_NN = 16
M_BLK = 256
TBL_W = 128

SC_CORES = 2
SC_SUBCORES = 16
SC_WORKERS = SC_CORES * SC_SUBCORES
SC_CHUNK = 128


def _stage_a_body(nblk, n, xyz_ref, xyzT_ref, feat_ref, wqT_ref, wkT_ref,
                  wvT_ref, knn_ref, q_ref, table_ref):
    b = pl.program_id(0)
    xb = xyz_ref[...]
    xT = xyzT_ref[0]
    fb = feat_ref[...]

    mm = jnp.dot(xb, xT, preferred_element_type=jnp.float32)
    rowsq = jnp.sum(xb * xb, axis=1, keepdims=True)
    colsq = jnp.sum(xT * xT, axis=0, keepdims=True)
    d = -2.0 * mm
    d = d + rowsq
    d = d + colsq

    m_rows = d.shape[0]
    ngrp = n // 128
    inf = jnp.float32(jnp.inf)
    big = jnp.int32(1 << 30)
    iota_l = lax.broadcasted_iota(jnp.int32, (m_rows, 128), 1)
    a1 = d[:, 0:128]
    c1 = iota_l
    a2 = jnp.full((m_rows, 128), inf, jnp.float32)
    c2 = jnp.full((m_rows, 128), big, jnp.int32)
    a3 = a2
    c3 = c2
    for s in range(1, ngrp):
        v = d[:, s * 128:(s + 1) * 128]
        cc = iota_l + s * 128
        l1 = v < a1
        l2 = v < a2
        l3 = v < a3
        a3n = jnp.where(l3, jnp.where(l2, a2, v), a3)
        c3n = jnp.where(l3, jnp.where(l2, c2, cc), c3)
        a2n = jnp.where(l2, jnp.where(l1, a1, v), a2)
        c2n = jnp.where(l2, jnp.where(l1, c1, cc), c2)
        a1 = jnp.where(l1, v, a1)
        c1 = jnp.where(l1, cc, c1)
        a2, a3, c2, c3 = a2n, a3n, c2n, c3n
    S = jnp.concatenate([a1, a2, a3], axis=1)
    CCf = jnp.concatenate([c1, c2, c3], axis=1).astype(jnp.float32)
    bigf = jnp.float32(1e9)
    cols = []
    for _ in range(K_NN):
        mn = jnp.min(S, axis=1, keepdims=True)
        colf = jnp.min(jnp.where(S <= mn, CCf, bigf), axis=1, keepdims=True)
        cols.append(colf)
        S = jnp.where(CCf == colf, inf, S)
    knn = jnp.concatenate(cols, axis=1).astype(jnp.int32)
    knn_ref[...] = knn + b * n

    q_ref[...] = jnp.dot(fb, wqT_ref[...], preferred_element_type=jnp.float32)
    kf = jnp.dot(fb, wkT_ref[...], preferred_element_type=jnp.float32)
    v = jnp.dot(fb, wvT_ref[...], preferred_element_type=jnp.float32)
    kfi = lax.bitcast_convert_type(
        kf.astype(jnp.bfloat16), jnp.uint16).astype(jnp.uint32)
    vi = lax.bitcast_convert_type(
        v.astype(jnp.bfloat16), jnp.uint16).astype(jnp.uint32)
    kv = lax.bitcast_convert_type(kfi | (vi << jnp.uint32(16)), jnp.float32)
    table_ref[...] = jnp.concatenate(
        [kv, xb, jnp.zeros((m_rows, 56), jnp.float32)], axis=1)


def _stage_a(xyz2p, xyzTp, feat2, wqT, wkT, wvT, B, N, dim):
    nblk = N // M_BLK
    grid = (B, nblk)
    bn = B * N
    row_spec = lambda w: pl.BlockSpec((M_BLK, w), lambda b, i: (b * nblk + i, 0))
    full2 = lambda a, c: pl.BlockSpec((a, c), lambda b, i: (0, 0))
    return pl.pallas_call(
        functools.partial(_stage_a_body, nblk, N),
        grid=grid,
        in_specs=[
            row_spec(8),
            pl.BlockSpec((1, 8, N), lambda b, i: (b, 0, 0)),
            row_spec(dim),
            full2(dim, dim), full2(dim, dim), full2(dim, dim),
        ],
        out_specs=[
            row_spec(K_NN),
            row_spec(dim),
            row_spec(TBL_W),
        ],
        out_shape=[
            jax.ShapeDtypeStruct((bn, K_NN), jnp.int32),
            jax.ShapeDtypeStruct((bn, dim), jnp.float32),
            jax.ShapeDtypeStruct((bn, TBL_W), jnp.float32),
        ],
    )(xyz2p, xyzTp, feat2, wqT, wkT, wvT)


def _sc_gather(table, idx_flat):
    tot = idx_flat.shape[0]
    per_w = tot // SC_WORKERS
    nch = per_w // SC_CHUNK
    mesh = plsc.VectorSubcoreMesh(core_axis_name="c", subcore_axis_name="s")

    @functools.partial(
        pl.kernel,
        mesh=mesh,
        out_type=jax.ShapeDtypeStruct((tot, TBL_W), jnp.float32),
        scratch_types=[
            pltpu.VMEM((per_w,), jnp.int32),
            pltpu.VMEM((SC_CHUNK, TBL_W), jnp.float32),
            pltpu.VMEM((SC_CHUNK, TBL_W), jnp.float32),
            pltpu.VMEM((SC_CHUNK, TBL_W), jnp.float32),
            pltpu.VMEM((SC_CHUNK, TBL_W), jnp.float32),
            pltpu.SemaphoreType.DMA,
            pltpu.SemaphoreType.DMA,
            pltpu.SemaphoreType.DMA,
            pltpu.SemaphoreType.DMA,
            pltpu.SemaphoreType.DMA,
        ],
    )
    def k(table_hbm, idx_hbm, out_hbm, idx_v,
          bufa, bufb, bufc, bufd, gsem, sa, sb, sc, sd):
        wid = lax.axis_index("s") * SC_CORES + lax.axis_index("c")
        base = wid * per_w
        pltpu.sync_copy(idx_hbm.at[pl.ds(base, per_w)], idx_v)

        def pair(i, c0, b0, s0, c1, b1, s1):
            o0 = pl.multiple_of(c0 * SC_CHUNK, 8)
            o1 = pl.multiple_of(c1 * SC_CHUNK, 8)
            d0 = out_hbm.at[pl.ds(base + o0, SC_CHUNK)]
            d1 = out_hbm.at[pl.ds(base + o1, SC_CHUNK)]

            @pl.when(i > 0)
            def _():
                pltpu.make_async_copy(b0, d0, s0).wait()
                pltpu.make_async_copy(b1, d1, s1).wait()

            h0 = pltpu.async_copy(
                table_hbm.at[idx_v.at[pl.ds(o0, SC_CHUNK)]], b0, gsem)
            h1 = pltpu.async_copy(
                table_hbm.at[idx_v.at[pl.ds(o1, SC_CHUNK)]], b1, gsem)
            h0.wait()
            h1.wait()
            pltpu.async_copy(b0, d0, s0)
            pltpu.async_copy(b1, d1, s1)

        def body(i, carry):
            pair(i, 4 * i, bufa, sa, 4 * i + 1, bufb, sb)
            pair(i, 4 * i + 2, bufc, sc, 4 * i + 3, bufd, sd)
            return carry

        lax.fori_loop(0, nch // 4, body, 0)
        last = (nch // 4 - 1) * 4 * SC_CHUNK
        for t, (b, s) in enumerate(((bufa, sa), (bufb, sb),
                                    (bufc, sc), (bufd, sd))):
            pltpu.make_async_copy(
                b, out_hbm.at[pl.ds(base + last + t * SC_CHUNK, SC_CHUNK)],
                s).wait()

    return k(table, idx_flat)


def _stage_c_body(g_ref, q_ref, xyzp_ref, feat_ref, pw1_ref, pb1_ref,
                  pw2_ref, pb2_ref, aw1_ref, ab1_ref, aw2_ref, ab2_ref,
                  lwT_ref, lb_ref, out_ref):
    m = q_ref.shape[0]
    dim = q_ref.shape[1]
    mk = m * K_NN
    g = g_ref[...]
    kv = lax.bitcast_convert_type(g[:, :, 0:dim], jnp.int32)
    kf = lax.bitcast_convert_type(
        lax.shift_left(kv, jnp.int32(16)), jnp.float32)
    v = lax.bitcast_convert_type(
        lax.bitwise_and(kv, jnp.int32(-65536)), jnp.float32)
    xg = g[:, :, dim:dim + 16]

    xc = xyzp_ref[...]
    bf = jnp.bfloat16
    rel = (xc[None, :, :] - xg).reshape(mk, 16)
    h = jnp.maximum(
        jnp.dot(rel, pw1_ref[...], preferred_element_type=jnp.float32)
        + pb1_ref[...], 0.0)
    pos = jnp.dot(h.astype(bf), pw2_ref[...].astype(bf),
                  preferred_element_type=jnp.float32) + pb2_ref[...]

    q = q_ref[...]
    pos3 = pos.reshape(K_NN, m, dim)
    a = (q[None, :, :] - kf + pos3).reshape(mk, dim)
    a = jnp.maximum(
        jnp.dot(a.astype(bf), aw1_ref[...].astype(bf),
                preferred_element_type=jnp.float32)
        + ab1_ref[...], 0.0)
    logits = (jnp.dot(a.astype(bf), aw2_ref[...].astype(bf),
                      preferred_element_type=jnp.float32)
              + ab2_ref[...]) * (1.0 / jnp.sqrt(jnp.float32(dim)))

    l3 = logits.reshape(K_NN, m, dim)
    mx = jnp.max(l3, axis=0, keepdims=True)
    e = jnp.exp(l3 - mx)
    s = jnp.sum(e, axis=0, keepdims=True)
    attn = e / s
    wv = v + pos3
    o = jnp.sum(attn * wv, axis=0)

    out_ref[...] = (jnp.dot(o, lwT_ref[...], preferred_element_type=jnp.float32)
                    + lb_ref[...]) + feat_ref[...]


def _stage_c(g, q2, xyzp, feat2, pw1, pb1, pw2, pb2, aw1, ab1, aw2, ab2,
             lwT, lb, dim):
    bn = q2.shape[0]
    nblk = bn // M_BLK
    attn_hid = aw1.shape[1]
    row = lambda w: pl.BlockSpec((M_BLK, w), lambda i: (i, 0))
    full = lambda a, c: pl.BlockSpec((a, c), lambda i: (0, 0))
    return pl.pallas_call(
        _stage_c_body,
        grid=(nblk,),
        in_specs=[
            pl.BlockSpec((K_NN, M_BLK, TBL_W), lambda i: (0, i, 0)),
            row(dim), row(16), row(dim),
            full(16, dim), full(1, dim),
            full(dim, dim), full(1, dim),
            full(dim, attn_hid), full(1, attn_hid),
            full(attn_hid, dim), full(1, dim),
            full(dim, dim), full(1, dim),
        ],
        out_specs=row(dim),
        out_shape=jax.ShapeDtypeStruct((bn, dim), jnp.float32),
    )(g, q2, xyzp, feat2, pw1, pb1, pw2, pb2, aw1, ab1, aw2, ab2, lwT, lb)


def kernel(xyz, feature, Wq, Wk, Wv, pe_w1, pe_b1, pe_w2, pe_b2,
           am_w1, am_b1, am_w2, am_b2, lf_w, lf_b):
    B, N, _ = xyz.shape
    dim = feature.shape[-1]
    bn = B * N

    xyz2 = xyz.reshape(bn, 3)
    xyz2p8 = jnp.pad(xyz2, ((0, 0), (0, 5)))
    xyzTp = jnp.pad(jnp.transpose(xyz, (0, 2, 1)), ((0, 0), (0, 5), (0, 0)))
    feat2 = feature.reshape(bn, dim)

    knn, q2, table = _stage_a(
        xyz2p8, xyzTp, feat2, Wq.T, Wk.T, Wv.T, B, N, dim)

    xyzp16 = jnp.pad(xyz2, ((0, 0), (0, 13)))
    g = _sc_gather(table, knn.T.reshape(bn * K_NN))
    g = g.reshape(K_NN, bn, TBL_W)

    pw1 = jnp.zeros((16, pe_w1.shape[0]), jnp.float32).at[:3, :].set(pe_w1.T)
    out2 = _stage_c(
        g, q2, xyzp16, feat2,
        pw1, pe_b1.reshape(1, -1), pe_w2.T, pe_b2.reshape(1, -1),
        am_w1.T, am_b1.reshape(1, -1), am_w2.T, am_b2.reshape(1, -1),
        lf_w.T, lf_b.reshape(1, -1), dim)
    return out2.reshape(B, N, dim)


# speedup vs baseline: 1.8089x; 1.0852x over previous
import functools

import jax
import jax.numpy as jnp
from jax import lax
from jax.experimental import pallas as pl
from jax.experimental.pallas import tpu as pltpu
from jax.experimental.pallas import tpu_sc as plsc

K_NN = 16
M_BLK = 256
TBL_W = 128

SC_CORES = 2
SC_SUBCORES = 16
SC_WORKERS = SC_CORES * SC_SUBCORES
SC_CHUNK = 128


def _stage_a_body(nblk, n, xyz_ref, xyzT_ref, feat_ref, wqT_ref, wkT_ref,
                  wvT_ref, knn_ref, q_ref, table_ref):
    b = pl.program_id(0)
    xb = xyz_ref[...]
    xT = xyzT_ref[0]
    fb = feat_ref[...]

    mm = jnp.dot(xb, xT, preferred_element_type=jnp.float32)
    rowsq = jnp.sum(xb * xb, axis=1, keepdims=True)
    colsq = jnp.sum(xT * xT, axis=0, keepdims=True)
    d = -2.0 * mm
    d = d + rowsq
    d = d + colsq

    m_rows = d.shape[0]
    ngrp = n // 128
    inf = jnp.float32(jnp.inf)
    big = jnp.int32(1 << 30)
    iota_l = lax.broadcasted_iota(jnp.int32, (m_rows, 128), 1)
    a1 = d[:, 0:128]
    c1 = iota_l
    a2 = jnp.full((m_rows, 128), inf, jnp.float32)
    c2 = jnp.full((m_rows, 128), big, jnp.int32)
    a3 = a2
    c3 = c2
    for s in range(1, ngrp):
        v = d[:, s * 128:(s + 1) * 128]
        cc = iota_l + s * 128
        l1 = v < a1
        l2 = v < a2
        l3 = v < a3
        a3n = jnp.where(l3, jnp.where(l2, a2, v), a3)
        c3n = jnp.where(l3, jnp.where(l2, c2, cc), c3)
        a2n = jnp.where(l2, jnp.where(l1, a1, v), a2)
        c2n = jnp.where(l2, jnp.where(l1, c1, cc), c2)
        a1 = jnp.where(l1, v, a1)
        c1 = jnp.where(l1, cc, c1)
        a2, a3, c2, c3 = a2n, a3n, c2n, c3n
    S = jnp.concatenate([a1, a2, a3], axis=1)
    CCf = jnp.concatenate([c1, c2, c3], axis=1).astype(jnp.float32)
    bigf = jnp.float32(1e9)
    cols = []
    for _ in range(K_NN):
        mn = jnp.min(S, axis=1, keepdims=True)
        colf = jnp.min(jnp.where(S <= mn, CCf, bigf), axis=1, keepdims=True)
        cols.append(colf)
        S = jnp.where(CCf == colf, inf, S)
    knn = jnp.concatenate(cols, axis=1).astype(jnp.int32)
    knn_ref[...] = knn + b * n

    q_ref[...] = jnp.dot(fb, wqT_ref[...], preferred_element_type=jnp.float32)
    kf = jnp.dot(fb, wkT_ref[...], preferred_element_type=jnp.float32)
    v = jnp.dot(fb, wvT_ref[...], preferred_element_type=jnp.float32)
    kfi = lax.bitcast_convert_type(
        kf.astype(jnp.bfloat16), jnp.uint16).astype(jnp.uint32)
    vi = lax.bitcast_convert_type(
        v.astype(jnp.bfloat16), jnp.uint16).astype(jnp.uint32)
    kv = lax.bitcast_convert_type(kfi | (vi << jnp.uint32(16)), jnp.float32)
    table_ref[...] = jnp.concatenate(
        [kv, xb, jnp.zeros((m_rows, 56), jnp.float32)], axis=1)


def _stage_a(xyz2p, xyzTp, feat2, wqT, wkT, wvT, B, N, dim):
    nblk = N // M_BLK
    grid = (B, nblk)
    bn = B * N
    row_spec = lambda w: pl.BlockSpec((M_BLK, w), lambda b, i: (b * nblk + i, 0))
    full2 = lambda a, c: pl.BlockSpec((a, c), lambda b, i: (0, 0))
    return pl.pallas_call(
        functools.partial(_stage_a_body, nblk, N),
        grid=grid,
        in_specs=[
            row_spec(8),
            pl.BlockSpec((1, 8, N), lambda b, i: (b, 0, 0)),
            row_spec(dim),
            full2(dim, dim), full2(dim, dim), full2(dim, dim),
        ],
        out_specs=[
            row_spec(K_NN),
            row_spec(dim),
            row_spec(TBL_W),
        ],
        out_shape=[
            jax.ShapeDtypeStruct((bn, K_NN), jnp.int32),
            jax.ShapeDtypeStruct((bn, dim), jnp.float32),
            jax.ShapeDtypeStruct((bn, TBL_W), jnp.float32),
        ],
    )(xyz2p, xyzTp, feat2, wqT, wkT, wvT)


def _sc_gather(table, idx_flat):
    tot = idx_flat.shape[0]
    per_w = tot // SC_WORKERS
    nch = per_w // SC_CHUNK
    mesh = plsc.VectorSubcoreMesh(core_axis_name="c", subcore_axis_name="s")

    @functools.partial(
        pl.kernel,
        mesh=mesh,
        out_type=jax.ShapeDtypeStruct((tot, TBL_W), jnp.float32),
        scratch_types=[
            pltpu.VMEM((per_w,), jnp.int32),
            pltpu.VMEM((SC_CHUNK, TBL_W), jnp.float32),
            pltpu.VMEM((SC_CHUNK, TBL_W), jnp.float32),
            pltpu.VMEM((SC_CHUNK, TBL_W), jnp.float32),
            pltpu.VMEM((SC_CHUNK, TBL_W), jnp.float32),
            pltpu.SemaphoreType.DMA,
            pltpu.SemaphoreType.DMA,
            pltpu.SemaphoreType.DMA,
            pltpu.SemaphoreType.DMA,
            pltpu.SemaphoreType.DMA,
        ],
    )
    def k(table_hbm, idx_hbm, out_hbm, idx_v,
          bufa, bufb, bufc, bufd, gsem, sa, sb, sc, sd):
        wid = lax.axis_index("s") * SC_CORES + lax.axis_index("c")
        base = wid * per_w
        pltpu.sync_copy(idx_hbm.at[pl.ds(base, per_w)], idx_v)

        def pair(i, c0, b0, s0, c1, b1, s1):
            o0 = pl.multiple_of(c0 * SC_CHUNK, 8)
            o1 = pl.multiple_of(c1 * SC_CHUNK, 8)
            d0 = out_hbm.at[pl.ds(base + o0, SC_CHUNK)]
            d1 = out_hbm.at[pl.ds(base + o1, SC_CHUNK)]

            @pl.when(i > 0)
            def _():
                pltpu.make_async_copy(b0, d0, s0).wait()
                pltpu.make_async_copy(b1, d1, s1).wait()

            h0 = pltpu.async_copy(
                table_hbm.at[idx_v.at[pl.ds(o0, SC_CHUNK)]], b0, gsem)
            h1 = pltpu.async_copy(
                table_hbm.at[idx_v.at[pl.ds(o1, SC_CHUNK)]], b1, gsem)
            h0.wait()
            h1.wait()
            pltpu.async_copy(b0, d0, s0)
            pltpu.async_copy(b1, d1, s1)

        def body(i, carry):
            pair(i, 4 * i, bufa, sa, 4 * i + 1, bufb, sb)
            pair(i, 4 * i + 2, bufc, sc, 4 * i + 3, bufd, sd)
            return carry

        lax.fori_loop(0, nch // 4, body, 0)
        last = (nch // 4 - 1) * 4 * SC_CHUNK
        for t, (b, s) in enumerate(((bufa, sa), (bufb, sb),
                                    (bufc, sc), (bufd, sd))):
            pltpu.make_async_copy(
                b, out_hbm.at[pl.ds(base + last + t * SC_CHUNK, SC_CHUNK)],
                s).wait()

    return k(table, idx_flat)


def _stage_c_body(g_ref, q_ref, xyzp_ref, feat_ref, pw1_ref, pb1_ref,
                  pw2_ref, pb2_ref, aw1_ref, ab1_ref, aw2_ref, ab2_ref,
                  lwT_ref, lb_ref, out_ref):
    m = q_ref.shape[0]
    dim = q_ref.shape[1]
    mk = m * K_NN
    g = g_ref[...]
    kv = lax.bitcast_convert_type(g[:, :, 0:dim], jnp.int32)
    kf = lax.bitcast_convert_type(
        lax.shift_left(kv, jnp.int32(16)), jnp.float32)
    v = lax.bitcast_convert_type(
        lax.bitwise_and(kv, jnp.int32(-65536)), jnp.float32)
    xg = g[:, :, dim:dim + 16]

    xc = xyzp_ref[...]
    bf = jnp.bfloat16
    rel = (xc[None, :, :] - xg).reshape(mk, 16)
    h = jnp.maximum(
        jnp.dot(rel, pw1_ref[...], preferred_element_type=jnp.float32)
        + pb1_ref[...], 0.0)
    pos = jnp.dot(h.astype(bf), pw2_ref[...].astype(bf),
                  preferred_element_type=jnp.float32) + pb2_ref[...]

    q = q_ref[...]
    pos3 = pos.reshape(K_NN, m, dim)
    a = (q[None, :, :] - kf + pos3).reshape(mk, dim)
    a = jnp.maximum(
        jnp.dot(a.astype(bf), aw1_ref[...].astype(bf),
                preferred_element_type=jnp.float32)
        + ab1_ref[...], 0.0)
    logits = (jnp.dot(a.astype(bf), aw2_ref[...].astype(bf),
                      preferred_element_type=jnp.float32)
              + ab2_ref[...]) * (1.0 / jnp.sqrt(jnp.float32(dim)))

    l3 = logits.reshape(K_NN, m, dim)
    mx = jnp.max(l3, axis=0, keepdims=True)
    e = jnp.exp(l3 - mx)
    s = jnp.sum(e, axis=0, keepdims=True)
    attn = e / s
    wv = v + pos3
    o = jnp.sum(attn * wv, axis=0)

    out_ref[...] = (jnp.dot(o, lwT_ref[...], preferred_element_type=jnp.float32)
                    + lb_ref[...]) + feat_ref[...]


def _stage_c(g, q2, xyzp, feat2, pw1, pb1, pw2, pb2, aw1, ab1, aw2, ab2,
             lwT, lb, dim):
    bn = q2.shape[0]
    nblk = bn // M_BLK
    attn_hid = aw1.shape[1]
    row = lambda w: pl.BlockSpec((M_BLK, w), lambda i: (i, 0))
    full = lambda a, c: pl.BlockSpec((a, c), lambda i: (0, 0))
    return pl.pallas_call(
        _stage_c_body,
        grid=(nblk,),
        in_specs=[
            pl.BlockSpec((K_NN, M_BLK, TBL_W), lambda i: (0, i, 0)),
            row(dim), row(16), row(dim),
            full(16, dim), full(1, dim),
            full(dim, dim), full(1, dim),
            full(dim, attn_hid), full(1, attn_hid),
            full(attn_hid, dim), full(1, dim),
            full(dim, dim), full(1, dim),
        ],
        out_specs=row(dim),
        out_shape=jax.ShapeDtypeStruct((bn, dim), jnp.float32),
    )(g, q2, xyzp, feat2, pw1, pb1, pw2, pb2, aw1, ab1, aw2, ab2, lwT, lb)


def kernel(xyz, feature, Wq, Wk, Wv, pe_w1, pe_b1, pe_w2, pe_b2,
           am_w1, am_b1, am_w2, am_b2, lf_w, lf_b):
    B, N, _ = xyz.shape
    dim = feature.shape[-1]
    bn = B * N

    xyz2 = xyz.reshape(bn, 3)
    xyz2p8 = jnp.pad(xyz2, ((0, 0), (0, 5)))
    xyzTp = jnp.pad(jnp.transpose(xyz, (0, 2, 1)), ((0, 0), (0, 5), (0, 0)))
    feat2 = feature.reshape(bn, dim)
    xyzp16 = jnp.pad(xyz2, ((0, 0), (0, 13)))
    pw1 = jnp.zeros((16, pe_w1.shape[0]), jnp.float32).at[:3, :].set(pe_w1.T)

    hb = B // 2
    hn = hb * N
    outs = []
    for h in range(2):
        r = slice(h * hn, (h + 1) * hn)
        knn, q2, table = _stage_a(
            xyz2p8[r], xyzTp[h * hb:(h + 1) * hb], feat2[r],
            Wq.T, Wk.T, Wv.T, hb, N, dim)
        g = _sc_gather(table, knn.T.reshape(hn * K_NN))
        g = g.reshape(K_NN, hn, TBL_W)
        outs.append(_stage_c(
            g, q2, xyzp16[r], feat2[r],
            pw1, pe_b1.reshape(1, -1), pe_w2.T, pe_b2.reshape(1, -1),
            am_w1.T, am_b1.reshape(1, -1), am_w2.T, am_b2.reshape(1, -1),
            lf_w.T, lf_b.reshape(1, -1), dim))
    return jnp.concatenate(outs, axis=0).reshape(B, N, dim)
